# baseline (speedup 1.0000x reference)
import functools
import math

import jax
import jax.numpy as jnp
from jax import lax
from jax.experimental import pallas as pl
from jax.experimental.pallas import tpu as pltpu
from jax.experimental.pallas import tpu_sc as plsc

_BETA = 1.0 - 0.001
_LOG_BETA = math.log(_BETA)
_NBINS = 32


def _make_sc_hist(b, h, w):
    info = plsc.get_sparse_core_info()
    nc, ns, lanes = info.num_cores, info.num_subcores, info.num_lanes
    nw = nc * ns
    slabs_per_b = nw // b
    rows = h // slabs_per_b
    assert b * h * w == nw * rows * w and w % lanes == 0

    mesh = plsc.VectorSubcoreMesh(core_axis_name="c", subcore_axis_name="s")

    @functools.partial(
        pl.kernel,
        mesh=mesh,
        out_type=jax.ShapeDtypeStruct((nw, _NBINS), jnp.float32),
        scratch_types=[
            pltpu.VMEM((rows, w), jnp.int32),
            pltpu.VMEM((4 * _NBINS * lanes,), jnp.float32),
            pltpu.VMEM((1, _NBINS), jnp.float32),
        ],
        compiler_params=pltpu.CompilerParams(needs_layout_passes=False),
        cost_estimate=pl.CostEstimate(
            flops=2 * b * h * w, bytes_accessed=4 * b * h * w,
            transcendentals=0),
    )
    def _sc_hist(t_hbm, out_hbm, lbl_v, bins_v, row_v):
        wid = lax.axis_index("s") * nc + lax.axis_index("c")
        batch = wid // slabs_per_b
        slab = wid % slabs_per_b
        pltpu.sync_copy(t_hbm.at[batch, pl.ds(slab * rows, rows)], lbl_v)

        zeros = jnp.zeros((lanes,), jnp.float32)
        for k in range(4 * _NBINS):
            bins_v[pl.ds(k * lanes, lanes)] = zeros

        lane_iota = lax.iota(jnp.int32, lanes)
        ones = jnp.ones((lanes,), jnp.float32)
        region = _NBINS * lanes

        def body(r, carry):
            for j in range(w // lanes):
                t_vec = lbl_v[r, pl.ds(j * lanes, lanes)]
                plsc.addupdate_scatter(
                    bins_v,
                    [t_vec * lanes + ((j % 4) * region + lane_iota)], ones)
            return carry

        lax.fori_loop(0, rows, body, 0)

        for half in range(_NBINS // lanes):
            acc = jnp.zeros((lanes,), jnp.float32)
            base = lane_iota * lanes + half * (lanes * lanes)
            for g in range(4):
                for l in range(lanes):
                    acc = acc + plsc.load_gather(
                        bins_v, [base + (g * region + l)])
            row_v[0, pl.ds(half * lanes, lanes)] = acc
        pltpu.sync_copy(row_v, out_hbm.at[pl.ds(wid, 1)])

    return _sc_hist


def _cbce_body(x_ref, t_ref, s_out_ref):
    step = pl.program_id(0) * pl.num_programs(1) + pl.program_id(1)

    x = x_ref[0]
    t = t_ref[...]

    m = jnp.max(x, axis=0, keepdims=True)
    s = jnp.sum(jnp.exp(x - m), axis=0, keepdims=True)
    lse = jnp.log(s) + m

    classes = jax.lax.broadcasted_iota(jnp.int32, x.shape, 0)
    onehot = classes == t
    contrib_s = jnp.sum(jnp.where(onehot, lse - x, 0.0), axis=(1, 2),
                        keepdims=True)

    @pl.when(step == 0)
    def _init():
        s_out_ref[...] = contrib_s

    @pl.when(step != 0)
    def _accum():
        s_out_ref[...] += contrib_s


def _combine_body(s_ref, h_ref, out_ref):
    s = s_ref[...]
    h = h_ref[...]
    n_row = jnp.sum(h, axis=0, keepdims=True)
    powb = jnp.exp(n_row * _LOG_BETA)
    w_row = jnp.where(n_row > 0.0, (1.0 - _BETA) / (1.0 - powb), 0.0)
    den = jnp.sum(w_row * n_row, keepdims=True)

    c = s.shape[0]
    prod = s[:, 0] * w_row
    i0 = jax.lax.broadcasted_iota(jnp.int32, (c, _NBINS), 0)
    i1 = jax.lax.broadcasted_iota(jnp.int32, (c, _NBINS), 1)
    num = jnp.sum(jnp.where(i0 == i1, prod, 0.0), keepdims=True)
    out_ref[...] = num / den


def kernel(logits, target):
    b, c, h, w = logits.shape
    sh = 128
    while h % sh != 0:
        sh //= 2
    nblk = h // sh

    hist = _make_sc_hist(b, h, w)(target)

    s_out = pl.pallas_call(
        _cbce_body,
        grid=(b, nblk),
        in_specs=[
            pl.BlockSpec((1, c, sh, w), lambda i, j: (i, 0, j, 0)),
            pl.BlockSpec((1, sh, w), lambda i, j: (i, j, 0)),
        ],
        out_specs=pl.BlockSpec((c, 1, 1), lambda i, j: (0, 0, 0)),
        out_shape=jax.ShapeDtypeStruct((c, 1, 1), jnp.float32),
        cost_estimate=pl.CostEstimate(
            flops=10 * b * c * h * w, bytes_accessed=4 * b * c * h * w,
            transcendentals=b * c * h * w),
    )(logits, target)

    out = pl.pallas_call(
        _combine_body,
        out_shape=jax.ShapeDtypeStruct((1, 1), jnp.float32),
    )(s_out, hist)
    return out[0, 0]


# speedup vs baseline: 1.0874x; 1.0874x over previous
import math

import jax
import jax.numpy as jnp
from jax.experimental import pallas as pl
from jax.experimental.pallas import tpu as pltpu

_BETA = 1.0 - 0.001
_LOG_BETA = math.log(_BETA)


def _cbce_body(x_ref, t_ref, out_ref, acc_s_ref, acc_n_ref):
    step = pl.program_id(0) * pl.num_programs(1) + pl.program_id(1)
    nsteps = pl.num_programs(0) * pl.num_programs(1)

    x = x_ref[0]
    t = t_ref[...]

    m = jnp.max(x, axis=0, keepdims=True)
    s = jnp.sum(jnp.exp(x - m), axis=0, keepdims=True)
    lse = jnp.log(s) + m

    classes = jax.lax.broadcasted_iota(jnp.int32, x.shape, 0)
    onehot = classes == t
    contrib_s = jnp.sum(jnp.where(onehot, lse - x, 0.0), axis=(1, 2),
                        keepdims=True)
    contrib_n = jnp.sum(jnp.where(onehot, 1.0, 0.0), axis=(1, 2),
                        keepdims=True)

    @pl.when(step == 0)
    def _init():
        acc_s_ref[...] = contrib_s
        acc_n_ref[...] = contrib_n

    @pl.when(step != 0)
    def _accum():
        acc_s_ref[...] += contrib_s
        acc_n_ref[...] += contrib_n

    @pl.when(step == nsteps - 1)
    def _epilogue():
        tv = acc_n_ref[...]
        powb = jnp.exp(tv * _LOG_BETA)
        w = jnp.where(tv > 0.0, (1.0 - _BETA) / (1.0 - powb), 0.0)
        num = jnp.sum(w * acc_s_ref[...], keepdims=True)
        den = jnp.sum(w * tv, keepdims=True)
        out_ref[...] = num / den


def kernel(logits, target):
    b, c, h, w = logits.shape
    sh = 128
    while h % sh != 0:
        sh //= 2
    nblk = h // sh

    out = pl.pallas_call(
        _cbce_body,
        grid=(b, nblk),
        in_specs=[
            pl.BlockSpec((1, c, sh, w), lambda i, j: (i, 0, j, 0)),
            pl.BlockSpec((1, sh, w), lambda i, j: (i, j, 0)),
        ],
        out_specs=pl.BlockSpec((1, 1, 1), lambda i, j: (0, 0, 0)),
        out_shape=jax.ShapeDtypeStruct((1, 1, 1), jnp.float32),
        scratch_shapes=[
            pltpu.VMEM((c, 1, 1), jnp.float32),
            pltpu.VMEM((c, 1, 1), jnp.float32),
        ],
    )(logits, target)
    return out[0, 0, 0]
